# baseline (device time: 168453 ns/iter reference)
import jax
import jax.numpy as jnp
from jax import lax
from jax.experimental import pallas as pl
from jax.experimental.pallas import tpu as pltpu

N_DEV = 4
B_LOC = 2
SQ = 512
HS = 256
HQ_LOC = 8
DH = 64
D_MODEL = 768
D_CHUNK = HQ_LOC * DH
BLK = 64
QSCALE = 0.125 * 1.4426950408889634


def kernel(x, Wq, K_ext, V_ext, Wo):
    xf = x.reshape(B_LOC * SQ, D_MODEL)

    def body(x_ref, wq_ref, k_hbm, v_hbm, wo_ref, out_ref,
             xbf, own_wq, own_wo, comm_wq, comm_wo, ctx_ref, kbuf, vbuf,
             send_wq, recv_wq, send_wo, recv_wo, kv_sems):
        my_pos = lax.axis_index("i")
        left = lax.rem(my_pos + N_DEV - 1, N_DEV)
        right = lax.rem(my_pos + 1, N_DEV)

        barrier = pltpu.get_barrier_semaphore()
        for nbr in (left, right):
            pl.semaphore_signal(barrier, inc=1, device_id=(nbr,),
                                device_id_type=pl.DeviceIdType.MESH)
        pl.semaphore_wait(barrier, 2)

        def issue_kv(step):
            j = lax.rem(my_pos + N_DEV - step, N_DEV)
            ops = []
            for b in range(B_LOC):
                bg = my_pos * B_LOC + b
                for h in range(HQ_LOC):
                    hg = j * HQ_LOC + h
                    for hbm, buf in ((k_hbm, kbuf), (v_hbm, vbuf)):
                        c = pltpu.make_async_copy(
                            hbm.at[bg, :, hg, :], buf.at[step, b, h],
                            kv_sems.at[step])
                        c.start()
                        ops.append(c)
            return ops

        kv_ops = [issue_kv(0)]

        xbf[...] = x_ref[...].astype(jnp.bfloat16)
        own_wq[...] = (wq_ref[...] * QSCALE).astype(jnp.bfloat16)
        own_wo[...] = wo_ref[...].astype(jnp.bfloat16)

        qb = lax.broadcasted_iota(jnp.int32, (SQ, SQ), 0) // BLK
        kb = lax.broadcasted_iota(jnp.int32, (SQ, SQ), 1) // BLK
        bias = jnp.where(kb <= qb, 0.0, -1e9).astype(jnp.float32)

        def hop_rdma(src, ref, sems_s, sems_r, h, tgt):
            return pltpu.make_async_remote_copy(
                src_ref=src, dst_ref=ref.at[h],
                send_sem=sems_s.at[h], recv_sem=sems_r.at[h],
                device_id=(tgt,), device_id_type=pl.DeviceIdType.MESH)

        def attention(step, wq_c):
            qf = lax.dot_general(
                xbf[...], wq_c, (((1,), (0,)), ((), ())),
                preferred_element_type=jnp.float32).astype(jnp.bfloat16)
            for b in range(B_LOC):
                for h in range(HQ_LOC):
                    kc = kbuf[step, b, h].astype(jnp.bfloat16)
                    vc = vbuf[step, b, h].astype(jnp.bfloat16)
                    q_bh = qf[b * SQ:(b + 1) * SQ, h * DH:(h + 1) * DH]
                    col = step * D_CHUNK + h * DH
                    for r0, r1, kv1 in ((0, HS, HS), (HS, SQ, SQ)):
                        s = lax.dot_general(
                            q_bh[r0:r1], kc[:kv1], (((1,), (1,)), ((), ())),
                            preferred_element_type=jnp.float32)
                        w = jnp.exp2(s + bias[r0:r1, :kv1])
                        wsum = jnp.sum(w, axis=1, keepdims=True)
                        c_bh = lax.dot_general(
                            w.astype(jnp.bfloat16), vc[:kv1],
                            (((1,), (0,)), ((), ())),
                            preferred_element_type=jnp.float32)
                        ctx_ref[b * SQ + r0:b * SQ + r1, col:col + DH] = (
                            c_bh / wsum).astype(jnp.bfloat16)

        def out_proj(slot, wo_c, first=False):
            contrib = lax.dot_general(
                ctx_ref[:, slot * D_CHUNK:(slot + 1) * D_CHUNK], wo_c,
                (((1,), (0,)), ((), ())),
                preferred_element_type=jnp.float32)
            if first:
                out_ref[...] = contrib
            else:
                out_ref[...] = out_ref[...] + contrib

        rdmas = []
        r_wq = hop_rdma(own_wq, comm_wq, send_wq, recv_wq, 0, right)
        r_wo = hop_rdma(own_wo, comm_wo, send_wo, recv_wo, 0, left)
        r_wq.start()
        r_wo.start()
        rdmas.append((r_wq, r_wo))

        for c in kv_ops[0]:
            c.wait()
        kv_ops.append(issue_kv(1))

        attention(0, own_wq[...])
        out_proj(0, own_wo[...], first=True)

        for s in range(1, N_DEV):
            r_wq, r_wo = rdmas[-1]
            r_wq.wait_recv()
            r_wo.wait_recv()
            if s < N_DEV - 1:
                n_wq = hop_rdma(comm_wq.at[s - 1], comm_wq, send_wq, recv_wq,
                                s, right)
                n_wo = hop_rdma(comm_wo.at[s - 1], comm_wo, send_wo, recv_wo,
                                s, left)
                n_wq.start()
                n_wo.start()
                rdmas.append((n_wq, n_wo))

            for c in kv_ops[s]:
                c.wait()
            if s < N_DEV - 1:
                kv_ops.append(issue_kv(s + 1))

            attention(s, comm_wq[s - 1])
            if s == 2:
                out_proj(2, comm_wo[1])
            elif s == 3:
                out_proj(3, comm_wo[0])
                out_proj(1, comm_wo[2])

        for r_wq, r_wo in rdmas:
            r_wq.wait_send()
            r_wo.wait_send()

    out = pl.pallas_call(
        body,
        out_shape=jax.ShapeDtypeStruct((B_LOC * SQ, D_MODEL), jnp.float32),
        in_specs=[
            pl.BlockSpec(memory_space=pltpu.VMEM),
            pl.BlockSpec(memory_space=pltpu.VMEM),
            pl.BlockSpec(memory_space=pl.ANY),
            pl.BlockSpec(memory_space=pl.ANY),
            pl.BlockSpec(memory_space=pltpu.VMEM),
        ],
        out_specs=pl.BlockSpec(memory_space=pltpu.VMEM),
        scratch_shapes=[
            pltpu.VMEM((B_LOC * SQ, D_MODEL), jnp.bfloat16),
            pltpu.VMEM((D_MODEL, D_CHUNK), jnp.bfloat16),
            pltpu.VMEM((D_CHUNK, D_MODEL), jnp.bfloat16),
            pltpu.VMEM((N_DEV - 1, D_MODEL, D_CHUNK), jnp.bfloat16),
            pltpu.VMEM((N_DEV - 1, D_CHUNK, D_MODEL), jnp.bfloat16),
            pltpu.VMEM((B_LOC * SQ, N_DEV * D_CHUNK), jnp.bfloat16),
            pltpu.VMEM((N_DEV, B_LOC, HQ_LOC, SQ, DH), jnp.float32),
            pltpu.VMEM((N_DEV, B_LOC, HQ_LOC, SQ, DH), jnp.float32),
            pltpu.SemaphoreType.DMA((N_DEV - 1,)),
            pltpu.SemaphoreType.DMA((N_DEV - 1,)),
            pltpu.SemaphoreType.DMA((N_DEV - 1,)),
            pltpu.SemaphoreType.DMA((N_DEV - 1,)),
            pltpu.SemaphoreType.DMA((N_DEV,)),
        ],
        compiler_params=pltpu.CompilerParams(
            collective_id=0, vmem_limit_bytes=100 * 1024 * 1024),
    )(xf, Wq, K_ext, V_ext, Wo)
    return out.reshape(B_LOC, SQ, D_MODEL)


# device time: 116066 ns/iter; 1.4514x vs baseline; 1.4514x over previous
import jax
import jax.numpy as jnp
from jax import lax
from jax.experimental import pallas as pl
from jax.experimental.pallas import tpu as pltpu

N_DEV = 4
B_LOC = 2
SQ = 512
HS = 256
HQ_LOC = 8
DH = 64
D_MODEL = 768
D_CHUNK = HQ_LOC * DH
BLK = 64
QSCALE = 0.125 * 1.4426950408889634


def kernel(x, Wq, K_ext, V_ext, Wo):
    xf = x.reshape(B_LOC * SQ, D_MODEL)
    K2 = K_ext.reshape(N_DEV * B_LOC, SQ, N_DEV * D_CHUNK)
    V2 = V_ext.reshape(N_DEV * B_LOC, SQ, N_DEV * D_CHUNK)

    def body(x_ref, wq_ref, k_hbm, v_hbm, wo_ref, out_ref,
             xbf, own_wq, own_wo, comm_wq, comm_wo, ctx_ref, kbuf, vbuf,
             send_wq, recv_wq, send_wo, recv_wo, kv_sems):
        my_pos = lax.axis_index("i")
        left = lax.rem(my_pos + N_DEV - 1, N_DEV)
        right = lax.rem(my_pos + 1, N_DEV)

        barrier = pltpu.get_barrier_semaphore()
        for nbr in (left, right):
            pl.semaphore_signal(barrier, inc=1, device_id=(nbr,),
                                device_id_type=pl.DeviceIdType.MESH)
        pl.semaphore_wait(barrier, 2)

        def issue_kv(step):
            j = lax.rem(my_pos + N_DEV - step, N_DEV)
            ops = []
            for b in range(B_LOC):
                bg = my_pos * B_LOC + b
                for hbm, buf in ((k_hbm, kbuf), (v_hbm, vbuf)):
                    c = pltpu.make_async_copy(
                        hbm.at[bg, :, pl.ds(j * D_CHUNK, D_CHUNK)],
                        buf.at[step, b], kv_sems.at[step])
                    c.start()
                    ops.append(c)
            return ops

        kv_ops = [issue_kv(0)]

        xbf[...] = x_ref[...].astype(jnp.bfloat16)
        own_wq[...] = (wq_ref[...] * QSCALE).astype(jnp.bfloat16)
        own_wo[...] = wo_ref[...].astype(jnp.bfloat16)

        qb = lax.broadcasted_iota(jnp.int32, (SQ, SQ), 0) // BLK
        kb = lax.broadcasted_iota(jnp.int32, (SQ, SQ), 1) // BLK
        bias = jnp.where(kb <= qb, 0.0, -1e9).astype(jnp.float32)

        def hop_rdma(src, ref, sems_s, sems_r, h, tgt):
            return pltpu.make_async_remote_copy(
                src_ref=src, dst_ref=ref.at[h],
                send_sem=sems_s.at[h], recv_sem=sems_r.at[h],
                device_id=(tgt,), device_id_type=pl.DeviceIdType.MESH)

        def attention(step, wq_c):
            qf = lax.dot_general(
                xbf[...], wq_c, (((1,), (0,)), ((), ())),
                preferred_element_type=jnp.float32).astype(jnp.bfloat16)
            for b in range(B_LOC):
                kc_all = kbuf[step, b].astype(jnp.bfloat16)
                vc_all = vbuf[step, b].astype(jnp.bfloat16)
                for h in range(HQ_LOC):
                    kc = kc_all[:, h * DH:(h + 1) * DH]
                    vc = vc_all[:, h * DH:(h + 1) * DH]
                    q_bh = qf[b * SQ:(b + 1) * SQ, h * DH:(h + 1) * DH]
                    col = step * D_CHUNK + h * DH
                    for r0, r1, kv1 in ((0, HS, HS), (HS, SQ, SQ)):
                        s = lax.dot_general(
                            q_bh[r0:r1], kc[:kv1], (((1,), (1,)), ((), ())),
                            preferred_element_type=jnp.float32)
                        w = jnp.exp2(s + bias[r0:r1, :kv1])
                        wsum = jnp.sum(w, axis=1, keepdims=True)
                        c_bh = lax.dot_general(
                            w.astype(jnp.bfloat16), vc[:kv1],
                            (((1,), (0,)), ((), ())),
                            preferred_element_type=jnp.float32)
                        ctx_ref[b * SQ + r0:b * SQ + r1, col:col + DH] = (
                            c_bh / wsum).astype(jnp.bfloat16)

        def out_proj(slot, wo_c, first=False):
            contrib = lax.dot_general(
                ctx_ref[:, slot * D_CHUNK:(slot + 1) * D_CHUNK], wo_c,
                (((1,), (0,)), ((), ())),
                preferred_element_type=jnp.float32)
            if first:
                out_ref[...] = contrib
            else:
                out_ref[...] = out_ref[...] + contrib

        rdmas = []
        r_wq = hop_rdma(own_wq, comm_wq, send_wq, recv_wq, 0, right)
        r_wo = hop_rdma(own_wo, comm_wo, send_wo, recv_wo, 0, left)
        r_wq.start()
        r_wo.start()
        rdmas.append((r_wq, r_wo))

        for c in kv_ops[0]:
            c.wait()
        kv_ops.append(issue_kv(1))

        attention(0, own_wq[...])
        out_proj(0, own_wo[...], first=True)

        for s in range(1, N_DEV):
            r_wq, r_wo = rdmas[-1]
            r_wq.wait_recv()
            r_wo.wait_recv()
            if s < N_DEV - 1:
                n_wq = hop_rdma(comm_wq.at[s - 1], comm_wq, send_wq, recv_wq,
                                s, right)
                n_wo = hop_rdma(comm_wo.at[s - 1], comm_wo, send_wo, recv_wo,
                                s, left)
                n_wq.start()
                n_wo.start()
                rdmas.append((n_wq, n_wo))

            for c in kv_ops[s]:
                c.wait()
            if s < N_DEV - 1:
                kv_ops.append(issue_kv(s + 1))

            attention(s, comm_wq[s - 1])
            if s == 2:
                out_proj(2, comm_wo[1])
            elif s == 3:
                out_proj(3, comm_wo[0])
                out_proj(1, comm_wo[2])

        for r_wq, r_wo in rdmas:
            r_wq.wait_send()
            r_wo.wait_send()

    out = pl.pallas_call(
        body,
        out_shape=jax.ShapeDtypeStruct((B_LOC * SQ, D_MODEL), jnp.float32),
        in_specs=[
            pl.BlockSpec(memory_space=pltpu.VMEM),
            pl.BlockSpec(memory_space=pltpu.VMEM),
            pl.BlockSpec(memory_space=pl.ANY),
            pl.BlockSpec(memory_space=pl.ANY),
            pl.BlockSpec(memory_space=pltpu.VMEM),
        ],
        out_specs=pl.BlockSpec(memory_space=pltpu.VMEM),
        scratch_shapes=[
            pltpu.VMEM((B_LOC * SQ, D_MODEL), jnp.bfloat16),
            pltpu.VMEM((D_MODEL, D_CHUNK), jnp.bfloat16),
            pltpu.VMEM((D_CHUNK, D_MODEL), jnp.bfloat16),
            pltpu.VMEM((N_DEV - 1, D_MODEL, D_CHUNK), jnp.bfloat16),
            pltpu.VMEM((N_DEV - 1, D_CHUNK, D_MODEL), jnp.bfloat16),
            pltpu.VMEM((B_LOC * SQ, N_DEV * D_CHUNK), jnp.bfloat16),
            pltpu.VMEM((N_DEV, B_LOC, SQ, D_CHUNK), jnp.float32),
            pltpu.VMEM((N_DEV, B_LOC, SQ, D_CHUNK), jnp.float32),
            pltpu.SemaphoreType.DMA((N_DEV - 1,)),
            pltpu.SemaphoreType.DMA((N_DEV - 1,)),
            pltpu.SemaphoreType.DMA((N_DEV - 1,)),
            pltpu.SemaphoreType.DMA((N_DEV - 1,)),
            pltpu.SemaphoreType.DMA((N_DEV,)),
        ],
        compiler_params=pltpu.CompilerParams(
            collective_id=0, vmem_limit_bytes=100 * 1024 * 1024),
    )(xf, Wq, K2, V2, Wo)
    return out.reshape(B_LOC, SQ, D_MODEL)
